# baseline (speedup 1.0000x reference)
import functools

import jax
import jax.numpy as jnp
from jax import lax
from jax.experimental import pallas as pl
from jax.experimental.pallas import tpu as pltpu
from jax.experimental.pallas import tpu_sc as plsc

_B, _L, _V, _H, _O = 1024, 50, 100000, 64, 10
_NW = 32
_BPW = (_B * _L) // _NW
_CHUNKS = 20
_CW = _BPW // _CHUNKS

_BBLK = 256
_GRID = _B // _BBLK


def _sc_gather(table2, idx3):
    mesh = plsc.VectorSubcoreMesh(core_axis_name="c", subcore_axis_name="s")

    @functools.partial(
        pl.kernel,
        mesh=mesh,
        out_type=jax.ShapeDtypeStruct((_B * _L, 2 * _H), jnp.float32),
        scratch_types=[
            pltpu.VMEM((_CHUNKS, _CW), jnp.int32),
            pltpu.VMEM((_CW, 2 * _H), jnp.float32),
            pltpu.VMEM((_CW, 2 * _H), jnp.float32),
            pltpu.SemaphoreType.DMA,
            pltpu.SemaphoreType.DMA,
            pltpu.SemaphoreType.DMA,
            pltpu.SemaphoreType.DMA,
        ],
    )
    def k(table_hbm, idx_hbm, out_hbm, idx_v, buf0, buf1, gs0, gs1, os0, os1):
        wid = lax.axis_index("s") * 2 + lax.axis_index("c")
        base = wid * _BPW
        pltpu.sync_copy(idx_hbm.at[wid], idx_v)
        bufs = (buf0, buf1)
        gsems = (gs0, gs1)
        osems = (os0, os1)
        puts = [None, None]
        for j in range(_CHUNKS):
            b = j % 2
            if puts[b] is not None:
                puts[b].wait()
            g = pltpu.async_copy(table_hbm.at[idx_v.at[j]], bufs[b], gsems[b])
            g.wait()
            puts[b] = pltpu.async_copy(
                bufs[b], out_hbm.at[pl.ds(base + j * _CW, _CW)], osems[b]
            )
        puts[0].wait()
        puts[1].wait()

    return k(table2, idx3)


def _tc_body(emb_ref, par_ref, lens_ref, wif_ref, whf_ref, bf_ref,
             wib_ref, whb_ref, bb_ref, wo_ref, bo_ref,
             out_ref, hc_ref, gxf_ref, gxb_ref):
    lens = lens_ref[...]
    whf = whf_ref[...]
    whb = whb_ref[...]

    emb = emb_ref[...]
    p = par_ref[...] != 0
    xs = jnp.where(p, emb[:, :, _H:2 * _H], emb[:, :, 0:_H])
    xs = xs.reshape(_L * _BBLK, _H)
    gxf_ref[...] = (jnp.dot(xs, wif_ref[...],
                            preferred_element_type=jnp.float32)
                    + bf_ref[...]).reshape(_L, _BBLK, 4 * _H)
    gxb_ref[...] = (jnp.dot(xs, wib_ref[...],
                            preferred_element_type=jnp.float32)
                    + bb_ref[...]).reshape(_L, _BBLK, 4 * _H)

    def cell(gx, h, c, whh):
        g = gx + jnp.dot(h, whh, preferred_element_type=jnp.float32)
        gi = jax.nn.sigmoid(g[:, 0:_H])
        gf = jax.nn.sigmoid(g[:, _H:2 * _H])
        gg = jnp.tanh(g[:, 2 * _H:3 * _H])
        go = jax.nn.sigmoid(g[:, 3 * _H:4 * _H])
        c_new = gf * c + gi * gg
        h_new = go * jnp.tanh(c_new)
        return h_new, c_new

    zeros = jnp.zeros((_BBLK, _H), jnp.float32)

    def step(s, carry):
        hf, cf, hb, cb = carry
        tb = _L - 1 - s
        hf_new, cf_new = cell(gxf_ref[s], hf, cf, whf)
        hb_new, cb_new = cell(gxb_ref[tb], hb, cb, whb)
        mf = lens > s
        mb = lens > tb
        hc_ref[s, :, 0:_H] = jnp.where(mf, hf_new, 0.0)
        hc_ref[tb, :, _H:2 * _H] = jnp.where(mb, hb_new, 0.0)
        return (jnp.where(mf, hf_new, hf), jnp.where(mf, cf_new, cf),
                jnp.where(mb, hb_new, hb), jnp.where(mb, cb_new, cb))

    lax.fori_loop(0, _L, step, (zeros, zeros, zeros, zeros))

    hcat = hc_ref[...].reshape(_L * _BBLK, 2 * _H)
    out_t = lax.dot_general(
        wo_ref[...], hcat, (((1,), (1,)), ((), ())),
        preferred_element_type=jnp.float32,
    ) + bo_ref[...]
    out_ref[...] = out_t.reshape(_O, _L, _BBLK)


def _tc_bilstm(emb_tm, par_tm, lens2, wif, whf, bf, wib, whb, bb, wo, bo):
    full = lambda shape: pl.BlockSpec(shape, lambda i: (0,) * len(shape))
    return pl.pallas_call(
        _tc_body,
        grid=(_GRID,),
        in_specs=[
            pl.BlockSpec((_L, _BBLK, 2 * _H), lambda i: (0, i, 0)),
            pl.BlockSpec((_L, _BBLK, 1), lambda i: (0, i, 0)),
            pl.BlockSpec((_BBLK, 1), lambda i: (i, 0)),
            full((_H, 4 * _H)), full((_H, 4 * _H)), full((1, 4 * _H)),
            full((_H, 4 * _H)), full((_H, 4 * _H)), full((1, 4 * _H)),
            full((_O, 2 * _H)), full((_O, 1)),
        ],
        out_specs=pl.BlockSpec((_O, _L, _BBLK), lambda i: (0, 0, i)),
        out_shape=jax.ShapeDtypeStruct((_O, _L, _B), jnp.float32),
        scratch_shapes=[
            pltpu.VMEM((_L, _BBLK, 2 * _H), jnp.float32),
            pltpu.VMEM((_L, _BBLK, 4 * _H), jnp.float32),
            pltpu.VMEM((_L, _BBLK, 4 * _H), jnp.float32),
        ],
        compiler_params=pltpu.CompilerParams(
            dimension_semantics=("parallel",),
        ),
    )(emb_tm, par_tm, lens2, wif, whf, bf, wib, whb, bb, wo, bo)


def kernel(x, batch_seq_len, table, W_ih_f, W_hh_f, b_ih_f, b_hh_f,
           W_ih_b, W_hh_b, b_ih_b, b_hh_b, W_out, b_out):
    table2 = table.reshape(_V // 2, 2 * _H)
    xt_flat = x.T.reshape(-1)
    idx3 = (xt_flat >> 1).reshape(_NW, _CHUNKS, _CW)
    par_tm = (xt_flat & 1).astype(jnp.int8).reshape(_L, _B, 1)

    emb_flat = _sc_gather(table2, idx3)
    emb_tm = emb_flat.reshape(_L, _B, 2 * _H)

    lens2 = batch_seq_len.astype(jnp.int32)[:, None]
    bf = (b_ih_f + b_hh_f)[None, :]
    bb = (b_ih_b + b_hh_b)[None, :]
    bo = b_out[:, None]

    out_olb = _tc_bilstm(emb_tm, par_tm, lens2, W_ih_f.T, W_hh_f.T, bf,
                         W_ih_b.T, W_hh_b.T, bb, W_out, bo)
    return jnp.transpose(out_olb, (2, 1, 0))
